# baseline (speedup 1.0000x reference)
import jax
import jax.numpy as jnp
from jax.experimental import pallas as pl

N, D, S = 16384, 256, 512
BN = 512
NB = N // BN


def _scores_kernel(x_ref, s_ref, scores_ref, m_ref, l_ref):
    k = pl.program_id(0)
    x = x_ref[...]
    s = s_ref[...]
    x_sq = jnp.sum(x * x, axis=1, keepdims=True)
    s_sq = jnp.sum(s * s, axis=1)[None, :]
    g = jax.lax.dot_general(x, s, (((1,), (1,)), ((), ())),
                            preferred_element_type=jnp.float32)
    d2 = jnp.maximum(x_sq - 2.0 * g + s_sq, 0.0)
    score = -1000.0 * jnp.sqrt(d2)
    scores_ref[...] = score

    tile_max = jnp.max(score, axis=0, keepdims=True)
    tile_sum = jnp.sum(jnp.exp(score - tile_max), axis=0, keepdims=True)

    @pl.when(k == 0)
    def _init():
        m_ref[...] = jnp.broadcast_to(tile_max, (8, S))
        l_ref[...] = jnp.broadcast_to(tile_sum, (8, S))

    @pl.when(k > 0)
    def _update():
        m_old = m_ref[...][0:1, :]
        l_old = l_ref[...][0:1, :]
        m_new = jnp.maximum(m_old, tile_max)
        l_new = (l_old * jnp.exp(m_old - m_new)
                 + tile_sum * jnp.exp(tile_max - m_new))
        m_ref[...] = jnp.broadcast_to(m_new, (8, S))
        l_ref[...] = jnp.broadcast_to(l_new, (8, S))


def _norm_kernel(scores_ref, m_ref, l_ref, out_ref):
    m = m_ref[...][0:1, :]
    l = l_ref[...][0:1, :]
    out_ref[...] = jnp.exp(scores_ref[...] - m) / l


def kernel(inputs, samples):
    scores, m, l = pl.pallas_call(
        _scores_kernel,
        grid=(NB,),
        in_specs=[
            pl.BlockSpec((BN, D), lambda k: (k, 0)),
            pl.BlockSpec((S, D), lambda k: (0, 0)),
        ],
        out_specs=[
            pl.BlockSpec((BN, S), lambda k: (k, 0)),
            pl.BlockSpec((8, S), lambda k: (0, 0)),
            pl.BlockSpec((8, S), lambda k: (0, 0)),
        ],
        out_shape=[
            jax.ShapeDtypeStruct((N, S), jnp.float32),
            jax.ShapeDtypeStruct((8, S), jnp.float32),
            jax.ShapeDtypeStruct((8, S), jnp.float32),
        ],
    )(inputs, samples)

    out = pl.pallas_call(
        _norm_kernel,
        grid=(NB,),
        in_specs=[
            pl.BlockSpec((BN, S), lambda k: (k, 0)),
            pl.BlockSpec((8, S), lambda k: (0, 0)),
            pl.BlockSpec((8, S), lambda k: (0, 0)),
        ],
        out_specs=pl.BlockSpec((BN, S), lambda k: (k, 0)),
        out_shape=jax.ShapeDtypeStruct((N, S), jnp.float32),
    )(scores, m, l)
    return out


# speedup vs baseline: 1.4278x; 1.4278x over previous
import jax
import jax.numpy as jnp
from jax.experimental import pallas as pl
from jax.experimental.pallas import tpu as pltpu

N, D, S = 16384, 256, 512
BN = 512
NB = N // BN


def _kernel(x_ref, s_ref, out_ref, scores_scr, m_scr, l_scr):
    p = pl.program_id(0)
    k = pl.program_id(1)

    @pl.when(p == 0)
    def _compute():
        x = x_ref[...]
        s = s_ref[...]
        x_sq = jnp.sum(x * x, axis=1, keepdims=True)
        s_sq = jnp.sum(s * s, axis=1)[None, :]
        g = jax.lax.dot_general(x, s, (((1,), (1,)), ((), ())),
                                preferred_element_type=jnp.float32)
        d2 = jnp.maximum(x_sq - 2.0 * g + s_sq, 0.0)
        score = -1000.0 * jnp.sqrt(d2)
        scores_scr[pl.ds(k * BN, BN), :] = score

        tile_max = jnp.max(score, axis=0, keepdims=True)
        tile_sum = jnp.sum(jnp.exp(score - tile_max), axis=0, keepdims=True)

        @pl.when(k == 0)
        def _init():
            m_scr[...] = jnp.broadcast_to(tile_max, (8, S))
            l_scr[...] = jnp.broadcast_to(tile_sum, (8, S))

        @pl.when(k > 0)
        def _update():
            m_old = m_scr[...][0:1, :]
            l_old = l_scr[...][0:1, :]
            m_new = jnp.maximum(m_old, tile_max)
            l_new = (l_old * jnp.exp(m_old - m_new)
                     + tile_sum * jnp.exp(tile_max - m_new))
            m_scr[...] = jnp.broadcast_to(m_new, (8, S))
            l_scr[...] = jnp.broadcast_to(l_new, (8, S))

    @pl.when(p == 1)
    def _normalize():
        m = m_scr[...][0:1, :]
        l = l_scr[...][0:1, :]
        out_ref[...] = jnp.exp(scores_scr[pl.ds(k * BN, BN), :] - m) / l


def kernel(inputs, samples):
    return pl.pallas_call(
        _kernel,
        grid=(2, NB),
        in_specs=[
            pl.BlockSpec((BN, D), lambda p, k: (k * (1 - p), 0)),
            pl.BlockSpec((S, D), lambda p, k: (0, 0)),
        ],
        out_specs=pl.BlockSpec((BN, S), lambda p, k: (k * p, 0)),
        out_shape=jax.ShapeDtypeStruct((N, S), jnp.float32),
        scratch_shapes=[
            pltpu.VMEM((N, S), jnp.float32),
            pltpu.VMEM((8, S), jnp.float32),
            pltpu.VMEM((8, S), jnp.float32),
        ],
    )(inputs, samples)


# speedup vs baseline: 1.4482x; 1.0142x over previous
import jax
import jax.numpy as jnp
from jax.experimental import pallas as pl
from jax.experimental.pallas import tpu as pltpu

N, D, S = 16384, 256, 512
BN = 512
NB = N // BN


def _kernel(x_ref, s_ref, out_ref, e_scr, mrun_scr, m_scr, l_scr, ssq_scr):
    p = pl.program_id(0)
    k = pl.program_id(1)

    @pl.when(p == 0)
    def _compute():
        @pl.when(k == 0)
        def _precompute_ssq():
            s = s_ref[...]
            s_sq = jnp.sum(s * s, axis=1)[None, :]
            ssq_scr[...] = jnp.broadcast_to(s_sq, (8, S))

        x = x_ref[...]
        x_sq = jnp.sum(x * x, axis=1, keepdims=True)
        s_sq = ssq_scr[...][0:1, :]
        g = jax.lax.dot_general(x, s_ref[...], (((1,), (1,)), ((), ())),
                                preferred_element_type=jnp.float32)
        d2 = jnp.maximum((x_sq - 2.0 * g) + s_sq, 0.0)
        score = -1000.0 * jnp.sqrt(d2)

        tile_max = jnp.max(score, axis=0, keepdims=True)

        @pl.when(k == 0)
        def _init():
            e = jnp.exp(score - tile_max)
            e_scr[pl.ds(0, BN), :] = e
            tile_sum = jnp.sum(e, axis=0, keepdims=True)
            m_scr[...] = jnp.broadcast_to(tile_max, (8, S))
            l_scr[...] = jnp.broadcast_to(tile_sum, (8, S))
            mrun_scr[pl.ds(0, 8), :] = jnp.broadcast_to(tile_max, (8, S))

        @pl.when(k > 0)
        def _update():
            m_old = m_scr[...][0:1, :]
            l_old = l_scr[...][0:1, :]
            m_new = jnp.maximum(m_old, tile_max)
            e = jnp.exp(score - m_new)
            e_scr[pl.ds(k * BN, BN), :] = e
            tile_sum = jnp.sum(e, axis=0, keepdims=True)
            l_new = l_old * jnp.exp(m_old - m_new) + tile_sum
            m_scr[...] = jnp.broadcast_to(m_new, (8, S))
            l_scr[...] = jnp.broadcast_to(l_new, (8, S))
            mrun_scr[pl.ds(k * 8, 8), :] = jnp.broadcast_to(m_new, (8, S))

    @pl.when(p == 1)
    def _normalize():
        m = m_scr[...][0:1, :]
        l = l_scr[...][0:1, :]
        m_k = mrun_scr[pl.ds(k * 8, 8), :][0:1, :]
        c = jnp.exp(m_k - m) / l
        out_ref[...] = e_scr[pl.ds(k * BN, BN), :] * c


def kernel(inputs, samples):
    return pl.pallas_call(
        _kernel,
        grid=(2, NB),
        in_specs=[
            pl.BlockSpec((BN, D), lambda p, k: (k * (1 - p), 0)),
            pl.BlockSpec((S, D), lambda p, k: (0, 0)),
        ],
        out_specs=pl.BlockSpec((BN, S), lambda p, k: (k * p, 0)),
        out_shape=jax.ShapeDtypeStruct((N, S), jnp.float32),
        scratch_shapes=[
            pltpu.VMEM((N, S), jnp.float32),
            pltpu.VMEM((NB * 8, S), jnp.float32),
            pltpu.VMEM((8, S), jnp.float32),
            pltpu.VMEM((8, S), jnp.float32),
            pltpu.VMEM((8, S), jnp.float32),
        ],
    )(inputs, samples)


# speedup vs baseline: 1.8888x; 1.3042x over previous
import jax
import jax.numpy as jnp
from jax.experimental import pallas as pl
from jax.experimental.pallas import tpu as pltpu

N, D, S = 16384, 256, 512
BN = 1024
NB = N // BN


def _kernel(x_ref, s_ref, out_ref, e_scr, mrun_scr, m_scr, l_scr, ssq_scr):
    p = pl.program_id(0)
    k = pl.program_id(1)

    @pl.when(p == 0)
    def _compute():
        @pl.when(k == 0)
        def _precompute_ssq():
            s = s_ref[...]
            s_sq = jnp.sum(s * s, axis=1)[None, :]
            ssq_scr[...] = jnp.broadcast_to(s_sq, (8, S))

        x = x_ref[...]
        x_sq = jnp.sum(x * x, axis=1, keepdims=True)
        s_sq = ssq_scr[...][0:1, :]
        g = jax.lax.dot_general(x, s_ref[...], (((1,), (1,)), ((), ())),
                                preferred_element_type=jnp.float32)
        d2 = jnp.maximum((x_sq - 2.0 * g) + s_sq, 0.0)
        score = -1000.0 * jnp.sqrt(d2)

        tile_max = jnp.max(score, axis=0, keepdims=True)

        @pl.when(k == 0)
        def _init():
            e = jnp.exp(score - tile_max)
            e_scr[pl.ds(0, BN), :] = e
            tile_sum = jnp.sum(e, axis=0, keepdims=True)
            m_scr[...] = jnp.broadcast_to(tile_max, (8, S))
            l_scr[...] = jnp.broadcast_to(tile_sum, (8, S))
            mrun_scr[pl.ds(0, 8), :] = jnp.broadcast_to(tile_max, (8, S))

        @pl.when(k > 0)
        def _update():
            m_old = m_scr[...][0:1, :]
            l_old = l_scr[...][0:1, :]
            m_new = jnp.maximum(m_old, tile_max)
            e = jnp.exp(score - m_new)
            e_scr[pl.ds(k * BN, BN), :] = e
            tile_sum = jnp.sum(e, axis=0, keepdims=True)
            l_new = l_old * jnp.exp(m_old - m_new) + tile_sum
            m_scr[...] = jnp.broadcast_to(m_new, (8, S))
            l_scr[...] = jnp.broadcast_to(l_new, (8, S))
            mrun_scr[pl.ds(k * 8, 8), :] = jnp.broadcast_to(m_new, (8, S))

    @pl.when(p == 1)
    def _normalize():
        m = m_scr[...][0:1, :]
        l = l_scr[...][0:1, :]
        m_k = mrun_scr[pl.ds(k * 8, 8), :][0:1, :]
        c = jnp.exp(m_k - m) / l
        out_ref[...] = e_scr[pl.ds(k * BN, BN), :] * c


def kernel(inputs, samples):
    return pl.pallas_call(
        _kernel,
        grid=(2, NB),
        in_specs=[
            pl.BlockSpec((BN, D), lambda p, k: (k * (1 - p), 0)),
            pl.BlockSpec((S, D), lambda p, k: (0, 0)),
        ],
        out_specs=pl.BlockSpec((BN, S), lambda p, k: (k * p, 0)),
        out_shape=jax.ShapeDtypeStruct((N, S), jnp.float32),
        scratch_shapes=[
            pltpu.VMEM((N, S), jnp.float32),
            pltpu.VMEM((NB * 8, S), jnp.float32),
            pltpu.VMEM((8, S), jnp.float32),
            pltpu.VMEM((8, S), jnp.float32),
            pltpu.VMEM((8, S), jnp.float32),
        ],
    )(inputs, samples)


# speedup vs baseline: 2.1370x; 1.1314x over previous
import jax
import jax.numpy as jnp
from jax.experimental import pallas as pl
from jax.experimental.pallas import tpu as pltpu

N, D, S = 16384, 256, 512
BN = 2048
NB = N // BN


def _kernel(x_ref, s_ref, out_ref, e_scr, mrun_scr, m_scr, l_scr, ssq_scr):
    p = pl.program_id(0)
    k = pl.program_id(1)

    @pl.when(p == 0)
    def _compute():
        @pl.when(k == 0)
        def _precompute_ssq():
            s = s_ref[...]
            s_sq = jnp.sum(s * s, axis=1)[None, :]
            ssq_scr[...] = jnp.broadcast_to(s_sq, (8, S))

        x = x_ref[...]
        x_sq = jnp.sum(x * x, axis=1, keepdims=True)
        s_sq = ssq_scr[...][0:1, :]
        g = jax.lax.dot_general(x, s_ref[...], (((1,), (1,)), ((), ())),
                                preferred_element_type=jnp.float32)
        d2 = jnp.maximum((x_sq - 2.0 * g) + s_sq, 0.0)
        score = -1000.0 * jnp.sqrt(d2)

        tile_max = jnp.max(score, axis=0, keepdims=True)

        @pl.when(k == 0)
        def _init():
            e = jnp.exp(score - tile_max)
            e_scr[pl.ds(0, BN), :] = e
            tile_sum = jnp.sum(e, axis=0, keepdims=True)
            m_scr[...] = jnp.broadcast_to(tile_max, (8, S))
            l_scr[...] = jnp.broadcast_to(tile_sum, (8, S))
            mrun_scr[pl.ds(0, 8), :] = jnp.broadcast_to(tile_max, (8, S))

        @pl.when(k > 0)
        def _update():
            m_old = m_scr[...][0:1, :]
            l_old = l_scr[...][0:1, :]
            m_new = jnp.maximum(m_old, tile_max)
            e = jnp.exp(score - m_new)
            e_scr[pl.ds(k * BN, BN), :] = e
            tile_sum = jnp.sum(e, axis=0, keepdims=True)
            l_new = l_old * jnp.exp(m_old - m_new) + tile_sum
            m_scr[...] = jnp.broadcast_to(m_new, (8, S))
            l_scr[...] = jnp.broadcast_to(l_new, (8, S))
            mrun_scr[pl.ds(k * 8, 8), :] = jnp.broadcast_to(m_new, (8, S))

    @pl.when(p == 1)
    def _normalize():
        m = m_scr[...][0:1, :]
        l = l_scr[...][0:1, :]
        m_k = mrun_scr[pl.ds(k * 8, 8), :][0:1, :]
        c = jnp.exp(m_k - m) / l
        out_ref[...] = e_scr[pl.ds(k * BN, BN), :] * c


def kernel(inputs, samples):
    return pl.pallas_call(
        _kernel,
        grid=(2, NB),
        in_specs=[
            pl.BlockSpec((BN, D), lambda p, k: (k * (1 - p), 0)),
            pl.BlockSpec((S, D), lambda p, k: (0, 0)),
        ],
        out_specs=pl.BlockSpec((BN, S), lambda p, k: (k * p, 0)),
        out_shape=jax.ShapeDtypeStruct((N, S), jnp.float32),
        scratch_shapes=[
            pltpu.VMEM((N, S), jnp.float32),
            pltpu.VMEM((NB * 8, S), jnp.float32),
            pltpu.VMEM((8, S), jnp.float32),
            pltpu.VMEM((8, S), jnp.float32),
            pltpu.VMEM((8, S), jnp.float32),
        ],
    )(inputs, samples)
